# baseline (speedup 1.0000x reference)
import functools

import jax
import jax.numpy as jnp
from jax import lax
from jax.experimental import pallas as pl
from jax.experimental.pallas import tpu as pltpu
from jax.experimental.pallas import tpu_sc as plsc

_HID = 64
_GAMMA = 12.0
_RHO = 0.4
_BATCH = 16384
_NENT = 100000
_NW = 32
_BPW = _BATCH // _NW
_CH = 128
_NCH = _BPW // _CH
_L = 16
_PREP_BLK = 12800


def _prep_body(e_ref, t_ref, r_ref, d_ref):
    comb = _RHO * e_ref[...] + (1.0 - _RHO) * t_ref[...]
    both = jnp.concatenate([comb, r_ref[...]], axis=0)
    d_ref[...] = both.T


def _score_body(h_hbm, r_hbm, t_hbm, d_hbm, out_hbm,
                hv, rv, tv, b0h, b0t, b0r, b1h, b1t, b1r,
                ov, sem0, sem1):
    wid = lax.axis_index("s") * 2 + lax.axis_index("c")
    base = wid * _BPW
    lane = lax.iota(jnp.int32, _L)
    bufsets = ((b0h, b0t, b0r), (b1h, b1t, b1r))
    sems = (sem0, sem1)

    pltpu.sync_copy(h_hbm.at[pl.ds(base, _BPW)], hv)
    pltpu.sync_copy(r_hbm.at[pl.ds(base, _BPW)], rv)
    pltpu.sync_copy(t_hbm.at[pl.ds(base, _BPW)], tv)

    def fire(c, s):
        bh, bt, br = bufsets[s]
        sl = pl.ds(c * _CH, _CH)
        return [
            pltpu.async_copy(d_hbm.at[hv.at[sl]], bh, sems[s]),
            pltpu.async_copy(d_hbm.at[tv.at[sl]], bt, sems[s]),
            pltpu.async_copy(d_hbm.at[rv.at[sl]], br, sems[s]),
        ]

    pend = [None, None]
    pend[0] = fire(0, 0)
    for c in range(_NCH):
        s = c % 2
        for cp in pend[s]:
            cp.wait()
        if c + 1 < _NCH:
            pend[1 - s] = fire(c + 1, 1 - s)
        bh, bt, br = bufsets[s]

        def grp_body(g, carry, bh=bh, bt=bt, br=br, c=c):
            svec = jnp.full((_L,), _GAMMA, jnp.float32)
            for jj in range(_L):
                j = g * _L + jj
                acc = jnp.zeros((_L,), jnp.float32)
                for k in range(_HID // _L):
                    sl = pl.ds(k * _L, _L)
                    slr = pl.ds(_HID + k * _L, _L)
                    x = bh[j, sl] - bt[j, sl] + br[j, slr]
                    acc = acc + jnp.abs(x)
                for sh in (8, 4, 2, 1):
                    acc = acc + acc.at[lane ^ sh].get(
                        mode="promise_in_bounds")
                svec = jnp.where(lane == jj, svec - acc, svec)
            ov[pl.ds(c * _CH + g * _L, _L)] = svec
            return carry

        lax.fori_loop(0, _CH // _L, grp_body, 0)

    pltpu.sync_copy(ov, out_hbm.at[pl.ds(base, _BPW)])


@jax.jit
def _run(h_idx, r_idx, t_idx, ent, rel, txt):
    d_table = pl.pallas_call(
        _prep_body,
        grid=(_NENT // _PREP_BLK + (1 if _NENT % _PREP_BLK else 0),),
        in_specs=[
            pl.BlockSpec((_HID, _PREP_BLK), lambda i: (0, i)),
            pl.BlockSpec((_HID, _PREP_BLK), lambda i: (0, i)),
            pl.BlockSpec((_HID, _PREP_BLK), lambda i: (0, i)),
        ],
        out_specs=pl.BlockSpec((_PREP_BLK, 2 * _HID), lambda i: (i, 0)),
        out_shape=jax.ShapeDtypeStruct((_NENT, 2 * _HID), jnp.float32),
    )(ent.T, txt.T, rel.T)

    mesh = plsc.VectorSubcoreMesh(core_axis_name="c", subcore_axis_name="s")
    f = functools.partial(
        pl.kernel,
        mesh=mesh,
        out_type=jax.ShapeDtypeStruct((_BATCH,), jnp.float32),
        scratch_types=(
            [pltpu.VMEM((_BPW,), jnp.int32)] * 3
            + [pltpu.VMEM((_CH, 2 * _HID), jnp.float32)] * 6
            + [pltpu.VMEM((_BPW,), jnp.float32)]
            + [pltpu.SemaphoreType.DMA] * 2
        ),
    )(_score_body)
    return f(h_idx, r_idx, t_idx, d_table)


def kernel(sample, entity_embedding_init, relation_embedding, entity_text_raw):
    st = sample.T
    h_idx = st[0]
    r_idx = st[1]
    t_idx = st[2]
    score = _run(h_idx, r_idx, t_idx,
                 entity_embedding_init, relation_embedding, entity_text_raw)
    return score[:, None]


# speedup vs baseline: 1.0422x; 1.0422x over previous
import functools

import jax
import jax.numpy as jnp
from jax import lax
from jax.experimental import pallas as pl
from jax.experimental.pallas import tpu as pltpu
from jax.experimental.pallas import tpu_sc as plsc

_HID = 64
_GAMMA = 12.0
_RHO = 0.4
_BATCH = 16384
_NENT = 100000
_NW = 32
_BPW = _BATCH // _NW
_CH = 64
_NCH = _BPW // _CH
_L = 16
_PREP_BLK = 12800


def _prep_body(e_ref, t_ref, r_ref, d_ref):
    comb = _RHO * e_ref[...] + (1.0 - _RHO) * t_ref[...]
    both = jnp.concatenate([comb, r_ref[...]], axis=0)
    d_ref[...] = both.T


def _score_body(s_hbm, d_hbm, out_hbm,
                sv, b0h, b0t, b0r, b1h, b1t, b1r, b2h, b2t, b2r,
                ov, sem0, sem1, sem2):
    wid = lax.axis_index("s") * 2 + lax.axis_index("c")
    base = wid * _BPW
    lane = lax.iota(jnp.int32, _L)
    bufsets = ((b0h, b0t, b0r), (b1h, b1t, b1r), (b2h, b2t, b2r))
    sems = (sem0, sem1, sem2)

    pltpu.sync_copy(s_hbm.at[:, pl.ds(base, _BPW)], sv)

    def fire(c, s):
        bh, bt, br = bufsets[s]
        sl = pl.ds(c * _CH, _CH)
        return [
            pltpu.async_copy(d_hbm.at[sv.at[0, sl]], bh, sems[s]),
            pltpu.async_copy(d_hbm.at[sv.at[2, sl]], bt, sems[s]),
            pltpu.async_copy(d_hbm.at[sv.at[1, sl]], br, sems[s]),
        ]

    _NBUF = 3
    pend = [None] * _NBUF
    pend[0] = fire(0, 0)
    pend[1] = fire(1, 1)
    for c in range(_NCH):
        s = c % _NBUF
        for cp in pend[s]:
            cp.wait()
        if c + 2 < _NCH:
            pend[(c + 2) % _NBUF] = fire(c + 2, (c + 2) % _NBUF)
        bh, bt, br = bufsets[s]

        def grp_body(g, carry, bh=bh, bt=bt, br=br, c=c):
            svec = jnp.full((_L,), _GAMMA, jnp.float32)
            for jj in range(_L):
                j = g * _L + jj
                acc = jnp.zeros((_L,), jnp.float32)
                for k in range(_HID // _L):
                    sl = pl.ds(k * _L, _L)
                    slr = pl.ds(_HID + k * _L, _L)
                    x = bh[j, sl] - bt[j, sl] + br[j, slr]
                    acc = acc + jnp.abs(x)
                for sh in (8, 4, 2, 1):
                    acc = acc + acc.at[lane ^ sh].get(
                        mode="promise_in_bounds")
                svec = jnp.where(lane == jj, svec - acc, svec)
            ov[pl.ds(c * _CH + g * _L, _L)] = svec
            return carry

        lax.fori_loop(0, _CH // _L, grp_body, 0)

    pltpu.sync_copy(ov, out_hbm.at[pl.ds(base, _BPW)])


@jax.jit
def _run(sample_t, ent, rel, txt):
    d_table = pl.pallas_call(
        _prep_body,
        grid=(_NENT // _PREP_BLK + (1 if _NENT % _PREP_BLK else 0),),
        in_specs=[
            pl.BlockSpec((_HID, _PREP_BLK), lambda i: (0, i)),
            pl.BlockSpec((_HID, _PREP_BLK), lambda i: (0, i)),
            pl.BlockSpec((_HID, _PREP_BLK), lambda i: (0, i)),
        ],
        out_specs=pl.BlockSpec((_PREP_BLK, 2 * _HID), lambda i: (i, 0)),
        out_shape=jax.ShapeDtypeStruct((_NENT, 2 * _HID), jnp.float32),
    )(ent.T, txt.T, rel.T)

    mesh = plsc.VectorSubcoreMesh(core_axis_name="c", subcore_axis_name="s")
    f = functools.partial(
        pl.kernel,
        mesh=mesh,
        out_type=jax.ShapeDtypeStruct((_BATCH,), jnp.float32),
        scratch_types=(
            [pltpu.VMEM((3, _BPW), jnp.int32)]
            + [pltpu.VMEM((_CH, 2 * _HID), jnp.float32)] * 9
            + [pltpu.VMEM((_BPW,), jnp.float32)]
            + [pltpu.SemaphoreType.DMA] * 3
        ),
    )(_score_body)
    return f(sample_t, d_table)


def kernel(sample, entity_embedding_init, relation_embedding, entity_text_raw):
    score = _run(sample.T,
                 entity_embedding_init, relation_embedding, entity_text_raw)
    return score[:, None]


# speedup vs baseline: 1.0430x; 1.0007x over previous
import functools

import jax
import jax.numpy as jnp
from jax import lax
from jax.experimental import pallas as pl
from jax.experimental.pallas import tpu as pltpu
from jax.experimental.pallas import tpu_sc as plsc

_HID = 64
_GAMMA = 12.0
_RHO = 0.4
_BATCH = 16384
_NENT = 100000
_NW = 32
_BPW = _BATCH // _NW
_CH = 64
_NCH = _BPW // _CH
_L = 16
_PREP_BLK = 12800


def _prep_body(e_ref, t_ref, r_ref, d_ref):
    comb = _RHO * e_ref[...] + (1.0 - _RHO) * t_ref[...]
    both = jnp.concatenate([comb, r_ref[...]], axis=0)
    d_ref[...] = both.T


def _score_body(s_hbm, d_hbm, out_hbm,
                sv, b0h, b0t, b0r, b1h, b1t, b1r, b2h, b2t, b2r,
                ov, sem0, sem1, sem2):
    wid = lax.axis_index("s") * 2 + lax.axis_index("c")
    base = wid * _BPW
    lane = lax.iota(jnp.int32, _L)
    bfly = [lane ^ sh for sh in (8, 4, 2, 1)]
    onehot = [lane == jj for jj in range(_L)]
    bufsets = ((b0h, b0t, b0r), (b1h, b1t, b1r), (b2h, b2t, b2r))
    sems = (sem0, sem1, sem2)

    pltpu.sync_copy(s_hbm.at[:, pl.ds(base, _BPW)], sv)

    def fire(c, s):
        bh, bt, br = bufsets[s]
        sl = pl.ds(c * _CH, _CH)
        return [
            pltpu.async_copy(d_hbm.at[sv.at[0, sl]], bh, sems[s]),
            pltpu.async_copy(d_hbm.at[sv.at[2, sl]], bt, sems[s]),
            pltpu.async_copy(d_hbm.at[sv.at[1, sl]], br, sems[s]),
        ]

    _NBUF = 3
    pend = [None] * _NBUF
    pend[0] = fire(0, 0)
    pend[1] = fire(1, 1)
    for c in range(_NCH):
        s = c % _NBUF
        for cp in pend[s]:
            cp.wait()
        if c + 2 < _NCH:
            pend[(c + 2) % _NBUF] = fire(c + 2, (c + 2) % _NBUF)
        bh, bt, br = bufsets[s]

        def grp_body(g, carry, bh=bh, bt=bt, br=br, c=c):
            svec = jnp.full((_L,), _GAMMA, jnp.float32)
            for jj in range(_L):
                j = g * _L + jj
                acc = jnp.zeros((_L,), jnp.float32)
                for k in range(_HID // _L):
                    sl = pl.ds(k * _L, _L)
                    slr = pl.ds(_HID + k * _L, _L)
                    x = bh[j, sl] - bt[j, sl] + br[j, slr]
                    acc = acc + jnp.abs(x)
                for idx in bfly:
                    acc = acc + acc.at[idx].get(mode="promise_in_bounds")
                svec = jnp.where(onehot[jj], svec - acc, svec)
            ov[pl.ds(c * _CH + g * _L, _L)] = svec
            return carry

        lax.fori_loop(0, _CH // _L, grp_body, 0)

    pltpu.sync_copy(ov, out_hbm.at[pl.ds(base, _BPW)])


@jax.jit
def _run(sample_t, ent, rel, txt):
    d_table = pl.pallas_call(
        _prep_body,
        grid=(_NENT // _PREP_BLK + (1 if _NENT % _PREP_BLK else 0),),
        in_specs=[
            pl.BlockSpec((_HID, _PREP_BLK), lambda i: (0, i)),
            pl.BlockSpec((_HID, _PREP_BLK), lambda i: (0, i)),
            pl.BlockSpec((_HID, _PREP_BLK), lambda i: (0, i)),
        ],
        out_specs=pl.BlockSpec((_PREP_BLK, 2 * _HID), lambda i: (i, 0)),
        out_shape=jax.ShapeDtypeStruct((_NENT, 2 * _HID), jnp.float32),
    )(ent.T, txt.T, rel.T)

    mesh = plsc.VectorSubcoreMesh(core_axis_name="c", subcore_axis_name="s")
    f = functools.partial(
        pl.kernel,
        mesh=mesh,
        out_type=jax.ShapeDtypeStruct((_BATCH,), jnp.float32),
        scratch_types=(
            [pltpu.VMEM((3, _BPW), jnp.int32)]
            + [pltpu.VMEM((_CH, 2 * _HID), jnp.float32)] * 9
            + [pltpu.VMEM((_BPW,), jnp.float32)]
            + [pltpu.SemaphoreType.DMA] * 3
        ),
    )(_score_body)
    return f(sample_t, d_table)


def kernel(sample, entity_embedding_init, relation_embedding, entity_text_raw):
    score = _run(sample.T,
                 entity_embedding_init, relation_embedding, entity_text_raw)
    return score[:, None]
